# baseline (speedup 1.0000x reference)
import jax
import jax.numpy as jnp
from jax import lax
from jax.experimental import pallas as pl
from jax.experimental.pallas import tpu as pltpu
from jax.experimental.pallas import tpu_sc as plsc

B = 16384
H = 200
D = 64
W = D // 2
NC = 2
NS = 16
NW = NC * NS
BPW = B // NW
GB = 64
H0 = 128
H1 = H - H0
RU = 8
NBUF = 16


def _issue(table_hbm, tok_v, rows_v, buf, g, sem):
    goff = pl.multiple_of(g * H, 8)
    pltpu.async_copy(
        table_hbm.at[tok_v.at[pl.ds(goff, H0)]],
        rows_v.at[buf, pl.ds(0, H0)], sem)
    pltpu.async_copy(
        table_hbm.at[tok_v.at[pl.ds(goff + H0, H1)]],
        rows_v.at[buf, pl.ds(H0, H1)], sem)


def _wait(table_hbm, rows_v, buf, sem):
    pltpu.make_async_copy(
        table_hbm.at[pl.ds(0, H)], rows_v.at[buf], sem).wait()


def _accum(rows_v, buf, out_v, g):
    zero = jnp.zeros((16,), jnp.float32)

    def rbody(r8, accs):
        a = list(accs)
        r0 = pl.multiple_of(r8 * RU, RU)
        for hlf in range(2):
            rows = [
                plsc.bitcast(rows_v[buf, r0 + j, pl.ds(16 * hlf, 16)],
                             jnp.bfloat16)
                for j in range(RU)]
            while len(rows) > 1:
                rows = [rows[i] + rows[i + 1] for i in range(0, len(rows), 2)]
            lo, hi = plsc.unpack(
                rows[0],
                format=plsc.PackFormat.INTERLEAVED,
                preferred_element_type=jnp.float32)
            a[2 * hlf] = a[2 * hlf] + lo
            a[2 * hlf + 1] = a[2 * hlf + 1] + hi
        return tuple(a)

    accs = lax.fori_loop(0, H // RU, rbody, (zero,) * 4)
    for c in range(4):
        out_v[g, pl.ds(16 * c, 16)] = accs[c]


def _body(tok_hbm, table_hbm, out_hbm, tok_v, rows_v, out_v, *sems):
    wid = lax.axis_index("s") * NC + lax.axis_index("c")
    base = wid * BPW

    def group(gi, _):
        gbase = pl.multiple_of(base + gi * GB, GB)
        pltpu.sync_copy(tok_hbm.at[pl.ds(gbase * H, GB * H)], tok_v)
        for b in range(NBUF - 1):
            _issue(table_hbm, tok_v, rows_v, b, b, sems[b])

        def quad(g4, _):
            g = g4 * NBUF
            for b in range(NBUF):
                _wait(table_hbm, rows_v, b, sems[b])
                _issue(table_hbm, tok_v, rows_v,
                       (b + NBUF - 1) % NBUF, g + b + NBUF - 1,
                       sems[(b + NBUF - 1) % NBUF])
                _accum(rows_v, b, out_v, g + b)
            return 0

        lax.fori_loop(0, GB // NBUF - 1, quad, 0)
        g = GB - NBUF
        for b in range(NBUF):
            bb = (g + b) % NBUF
            _wait(table_hbm, rows_v, bb, sems[bb])
            if b == 0:
                _issue(table_hbm, tok_v, rows_v,
                       (GB - 1) % NBUF, GB - 1, sems[(GB - 1) % NBUF])
            _accum(rows_v, bb, out_v, g + b)
        pltpu.sync_copy(out_v, out_hbm.at[pl.ds(gbase, GB)])
        return 0

    lax.fori_loop(0, BPW // GB, group, 0)


def _pack_table(emb_table):
    V = emb_table.shape[0]
    t = emb_table.astype(jnp.bfloat16).reshape(V, 2, 2, 16)
    t = t.transpose(0, 1, 3, 2)
    return jax.lax.bitcast_convert_type(t.reshape(V, W, 2), jnp.int32)


def kernel(x_tokens, x_lens, emb_table):
    del x_lens
    mesh = plsc.VectorSubcoreMesh(
        core_axis_name="c", subcore_axis_name="s",
        num_cores=NC, num_subcores=NS)
    run = pl.kernel(
        _body,
        out_type=jax.ShapeDtypeStruct((B, D), jnp.float32),
        mesh=mesh,
        scratch_types=[
            pltpu.VMEM((GB * H,), jnp.int32),
            pltpu.VMEM((NBUF, H, W), jnp.int32),
            pltpu.VMEM((GB, D), jnp.float32),
        ] + [pltpu.SemaphoreType.DMA] * NBUF,
        compiler_params=pltpu.CompilerParams(
            use_tc_tiling_on_sc=False, needs_layout_passes=False),
    )
    out = run(x_tokens.astype(jnp.int32).reshape(B * H), _pack_table(emb_table))
    return out


# speedup vs baseline: 1.0407x; 1.0407x over previous
import jax
import jax.numpy as jnp
from jax import lax
from jax.experimental import pallas as pl
from jax.experimental.pallas import tpu as pltpu
from jax.experimental.pallas import tpu_sc as plsc

B = 16384
H = 200
D = 64
W = D // 2
NC = 2
NS = 16
NW = NC * NS
BPW = B // NW
GB = 64
H0 = 128
H1 = H - H0
RU = 8
NBUF = 8


def _issue(table_hbm, tok_v, rows_v, buf, g, sem):
    goff = pl.multiple_of(g * H, 8)
    pltpu.async_copy(
        table_hbm.at[tok_v.at[pl.ds(goff, H0)]],
        rows_v.at[buf, pl.ds(0, H0)], sem)
    pltpu.async_copy(
        table_hbm.at[tok_v.at[pl.ds(goff + H0, H1)]],
        rows_v.at[buf, pl.ds(H0, H1)], sem)


def _wait(table_hbm, rows_v, buf, sem):
    pltpu.make_async_copy(
        table_hbm.at[pl.ds(0, H)], rows_v.at[buf], sem).wait()


def _accum(rows_v, buf, out_v, g):
    zero = jnp.zeros((16,), jnp.float32)

    def rbody(r8, accs):
        a = list(accs)
        r0 = pl.multiple_of(r8 * RU, RU)
        for hlf in range(2):
            rows = [
                plsc.bitcast(rows_v[buf, r0 + j, pl.ds(16 * hlf, 16)],
                             jnp.bfloat16)
                for j in range(RU)]
            while len(rows) > 1:
                rows = [rows[i] + rows[i + 1] for i in range(0, len(rows), 2)]
            lo, hi = plsc.unpack(
                rows[0],
                format=plsc.PackFormat.INTERLEAVED,
                preferred_element_type=jnp.float32)
            a[2 * hlf] = a[2 * hlf] + lo
            a[2 * hlf + 1] = a[2 * hlf + 1] + hi
        return tuple(a)

    accs = lax.fori_loop(0, H // RU, rbody, (zero,) * 4)
    for c in range(4):
        out_v[g, pl.ds(16 * c, 16)] = accs[c]


def _body(tok_hbm, table_hbm, out_hbm, tok_v, rows_v, out_v, *sems):
    wid = lax.axis_index("s") * NC + lax.axis_index("c")
    base = wid * BPW

    def group(gi, _):
        gbase = pl.multiple_of(base + gi * GB, GB)
        pltpu.sync_copy(tok_hbm.at[pl.ds(gbase * H, GB * H)], tok_v)
        for b in range(NBUF - 1):
            _issue(table_hbm, tok_v, rows_v, b, b, sems[b])

        def quad(g4, _):
            g = g4 * NBUF
            for b in range(NBUF):
                _wait(table_hbm, rows_v, b, sems[b])
                _issue(table_hbm, tok_v, rows_v,
                       (b + NBUF - 1) % NBUF, g + b + NBUF - 1,
                       sems[(b + NBUF - 1) % NBUF])
                _accum(rows_v, b, out_v, g + b)
            return 0

        lax.fori_loop(0, GB // NBUF - 1, quad, 0)
        g = GB - NBUF
        for b in range(NBUF):
            bb = (g + b) % NBUF
            _wait(table_hbm, rows_v, bb, sems[bb])
            if b == 0:
                _issue(table_hbm, tok_v, rows_v,
                       (GB - 1) % NBUF, GB - 1, sems[(GB - 1) % NBUF])
            _accum(rows_v, bb, out_v, g + b)
        pltpu.sync_copy(out_v, out_hbm.at[pl.ds(gbase, GB)])
        return 0

    lax.fori_loop(0, BPW // GB, group, 0)


def _pack_table(emb_table):
    V = emb_table.shape[0]
    t = emb_table.astype(jnp.bfloat16).reshape(V, 2, 2, 16)
    t = t.transpose(0, 1, 3, 2)
    return jax.lax.bitcast_convert_type(t.reshape(V, W, 2), jnp.int32)


def kernel(x_tokens, x_lens, emb_table):
    del x_lens
    mesh = plsc.VectorSubcoreMesh(
        core_axis_name="c", subcore_axis_name="s",
        num_cores=NC, num_subcores=NS)
    run = pl.kernel(
        _body,
        out_type=jax.ShapeDtypeStruct((B, D), jnp.float32),
        mesh=mesh,
        scratch_types=[
            pltpu.VMEM((GB * H,), jnp.int32),
            pltpu.VMEM((NBUF, H, W), jnp.int32),
            pltpu.VMEM((GB, D), jnp.float32),
        ] + [pltpu.SemaphoreType.DMA] * NBUF,
        compiler_params=pltpu.CompilerParams(
            use_tc_tiling_on_sc=False, needs_layout_passes=False),
    )
    out = run(x_tokens.astype(jnp.int32).reshape(B * H), _pack_table(emb_table))
    return out
